# Initial kernel scaffold; baseline (speedup 1.0000x reference)
import jax
import jax.numpy as jnp
from jax.experimental import pallas as pl


def kernel(x_student, x_question, x_concept, prev_h, e_att_src, e_att_dst, e_tag_src, e_tag_dst, e_pre_src, e_pre_dst, e_mas_src, e_mas_dst, label_src, label_dst, params):
    raise NotImplementedError("write your pallas kernel here")


# speedup vs baseline: 21.5695x; 21.5695x over previous
import functools

import jax
import jax.numpy as jnp
from jax import lax
from jax.experimental import pallas as pl
from jax.experimental.pallas import tpu as pltpu
from jax.experimental.pallas import tpu_sc as plsc

NCORE = 2
NSUB = 16
LANE = 16
B = 160
H = 32
AUGW = 48


def _cdiv(a, b):
    return (a + b - 1) // b


@functools.lru_cache(maxsize=None)
def _edge_kernel(e_pad, e_real, n_src, half, split):
    P = _cdiv(half, 2048) * 2048
    c16 = P // 16
    assert c16 % B == 0
    zcnt = c16 // B
    nblk = (e_pad // (NSUB if split else NSUB * NCORE)) // B
    mesh = plsc.VectorSubcoreMesh(core_axis_name="c", subcore_axis_name="s")

    def body(aug_hbm, ad_hbm, src_hbm, dst_hbm, acc_hbm, den_hbm,
             acc_sh, den_sh, sidx, didx, rows, msg, adv, exv, dstl, sem):
        c = lax.axis_index("c")
        s = lax.axis_index("s")
        lo = c * half if split else 0
        lanes = lax.iota(jnp.int32, 16)
        col_as = jnp.full((16,), H, jnp.int32)
        zv = jnp.zeros((16,), jnp.float32)

        def z2(i, _):
            msg[i // 2, pl.ds((i % 2) * 16, 16)] = zv
            return _
        lax.fori_loop(0, B * 2, z2, None)

        def z1(i, _):
            exv[pl.ds(i * 16, 16)] = zv
            return _
        lax.fori_loop(0, B // 16, z1, None)

        for k in range(zcnt):
            r0 = pl.multiple_of(s * c16 + k * B, 8)
            pltpu.sync_copy(msg, acc_sh.at[pl.ds(r0, B)])
            pltpu.sync_copy(exv, den_sh.at[pl.ds(r0, B)])
        plsc.subcore_barrier()

        base = (s * (e_pad // NSUB)) if split else (
            (s * NCORE + c) * (e_pad // (NSUB * NCORE)))

        def blk(b, _):
            e0 = pl.multiple_of(base + b * B, B)
            pltpu.sync_copy(src_hbm.at[pl.ds(e0, B)], sidx)
            pltpu.sync_copy(dst_hbm.at[pl.ds(e0, B)], didx)
            pltpu.async_copy(aug_hbm.at[sidx], rows, sem).wait()
            pltpu.async_copy(ad_hbm.at[didx], adv, sem).wait()

            def chunk(j, _):
                d16 = didx[pl.ds(j * 16, 16)]
                gidx = e0 + j * 16 + lanes
                valid = (gidx < e_real) & (d16 >= lo) & (d16 < lo + half)
                dl = jnp.where(valid, d16 - lo, 0)
                ad16 = adv[pl.ds(j * 16, 16)]
                as16 = plsc.load_gather(rows, [j * 16 + lanes, col_as])
                z = as16 + ad16
                e = jnp.maximum(z, 0.2 * z)
                ex = jnp.where(valid, jnp.exp(e), 0.0)
                exv[pl.ds(j * 16, 16)] = ex
                dstl[pl.ds(j * 16, 16)] = dl
                return _
            lax.fori_loop(0, B // 16, chunk, None)

            def scale(i, _):
                i16 = jnp.full((16,), i, jnp.int32)
                exb = plsc.load_gather(exv, [i16])
                r0 = plsc.load_gather(rows, [i16, lanes])
                r1 = plsc.load_gather(rows, [i16, lanes + 16])
                plsc.store_scatter(msg, [i16, lanes], r0 * exb)
                plsc.store_scatter(msg, [i16, lanes + 16], r1 * exb)
                return _
            lax.fori_loop(0, B, scale, None)

            pltpu.sync_copy(msg, acc_sh.at[dstl], add=True)
            pltpu.sync_copy(exv, den_sh.at[dstl], add=True)
            return _
        lax.fori_loop(0, nblk, blk, None)

        plsc.subcore_barrier()
        for k in range(zcnt):
            r0 = pl.multiple_of(s * c16 + k * B, 8)
            pltpu.sync_copy(acc_sh.at[pl.ds(r0, B)],
                            acc_hbm.at[c, pl.ds(r0, B)])
            d0 = pl.multiple_of(c * P + s * c16 + k * B, 8)
            pltpu.sync_copy(den_sh.at[pl.ds(r0, B)], den_hbm.at[pl.ds(d0, B)])

    return pl.kernel(
        body,
        mesh=mesh,
        compiler_params=pltpu.CompilerParams(
            needs_layout_passes=False, use_tc_tiling_on_sc=False),
        out_type=(jax.ShapeDtypeStruct((NCORE, P, H), jnp.float32),
                  jax.ShapeDtypeStruct((NCORE * P,), jnp.float32)),
        scratch_types=[
            pltpu.VMEM_SHARED((P, H), jnp.float32),
            pltpu.VMEM_SHARED((P,), jnp.float32),
            pltpu.VMEM((B,), jnp.int32),
            pltpu.VMEM((B,), jnp.int32),
            pltpu.VMEM((B, AUGW), jnp.float32),
            pltpu.VMEM((B, H), jnp.float32),
            pltpu.VMEM((B,), jnp.float32),
            pltpu.VMEM((B,), jnp.float32),
            pltpu.VMEM((B,), jnp.int32),
            pltpu.SemaphoreType.DMA,
        ],
    )


def _pad_edges(src, dst):
    e = src.shape[0]
    e_pad = _cdiv(e, NSUB * NCORE * B) * (NSUB * NCORE * B)
    if e_pad != e:
        pad = e_pad - e
        src = jnp.concatenate([src, jnp.zeros((pad,), src.dtype)])
        dst = jnp.concatenate([dst, jnp.zeros((pad,), dst.dtype)])
    return src, dst, e


def _gat_sc(p, x_src, x_dst, src_p, dst_p, e_real, n_dst):
    n_src = x_src.shape[0]
    hs = x_src @ p["W_src"]
    a_s = hs @ p["a_src"]
    aug = jnp.concatenate(
        [hs, a_s[:, None], jnp.zeros((n_src, AUGW - H - 1), jnp.float32)], 1)
    ad = x_dst @ (p["W_dst"] @ p["a_dst"])
    split = n_dst * H * 4 > 7 * 2**20
    half = n_dst // NCORE if split else n_dst
    P = _cdiv(half, 2048) * 2048
    k = _edge_kernel(src_p.shape[0], e_real, n_src, half, split)
    acc, den = k(aug, ad, src_p, dst_p)
    den = den.reshape(NCORE, P)
    if split:
        out = acc[:, :half].reshape(n_dst, H)
        d = den[:, :half].reshape(n_dst)
    else:
        out = (acc[0] + acc[1])[:half]
        d = (den[0] + den[1])[:half]
    return out / (d[:, None] + 1e-16) + p["bias"]


def _gru(p, x, h):
    gi = x @ p["W_ih"] + p["b_ih"]
    gh = h @ p["W_hh"] + p["b_hh"]
    ir, iz, inn = jnp.split(gi, 3, axis=1)
    hr, hz, hn = jnp.split(gh, 3, axis=1)
    r = jax.nn.sigmoid(ir + hr)
    z = jax.nn.sigmoid(iz + hz)
    n = jnp.tanh(inn + r * hn)
    return (1.0 - z) * n + z * h


def _mlp(p, x):
    return jax.nn.relu(x @ p["l1"]["W"] + p["l1"]["b"]) @ p["l2"]["W"] + p["l2"]["b"]


def kernel(x_student, x_question, x_concept, prev_h, e_att_src, e_att_dst,
           e_tag_src, e_tag_dst, e_pre_src, e_pre_dst, e_mas_src, e_mas_dst,
           label_src, label_dst, params):
    ns, nq, nc = x_student.shape[0], x_question.shape[0], x_concept.shape[0]
    att_s, att_d, att_e = _pad_edges(e_att_src, e_att_dst)
    tag_s, tag_d, tag_e = _pad_edges(e_tag_src, e_tag_dst)
    pre_s, pre_d, pre_e = _pad_edges(e_pre_src, e_pre_dst)
    mas_s, mas_d, mas_e = _pad_edges(e_mas_src, e_mas_dst)

    xs = x_student @ params["enc_s"]["W"] + params["enc_s"]["b"]
    xq = x_question @ params["enc_q"]["W"] + params["enc_q"]["b"]
    xc = x_concept @ params["enc_c"]["W"] + params["enc_c"]["b"]
    xs = _gru(params["gru"], xs, prev_h)

    def hetero(conv, xs, xq, xc):
        s = (_gat_sc(conv["rev_att"], xq, xs, att_d, att_s, att_e, ns)
             + _gat_sc(conv["rev_mas"], xc, xs, mas_d, mas_s, mas_e, ns))
        q = (_gat_sc(conv["att"], xs, xq, att_s, att_d, att_e, nq)
             + _gat_sc(conv["rev_tag"], xc, xq, tag_d, tag_s, tag_e, nq))
        c = (_gat_sc(conv["tag"], xq, xc, tag_s, tag_d, tag_e, nc)
             + _gat_sc(conv["pre"], xc, xc, pre_s, pre_d, pre_e, nc)
             + _gat_sc(conv["mas"], xs, xc, mas_s, mas_d, mas_e, nc))
        return s, q, c

    xs, xq, xc = hetero(params["conv1"], xs, xq, xc)
    xs, xq, xc = jax.nn.relu(xs), jax.nn.relu(xq), jax.nn.relu(xc)
    xs, xq, xc = hetero(params["conv2"], xs, xq, xc)
    xs, xq, xc = jax.nn.relu(xs), jax.nn.relu(xq), jax.nn.relu(xc)

    pair = jnp.concatenate([xs[label_src], xq[label_dst]], axis=-1)
    logits = _mlp(params["head_corr"], pair)[:, 0]
    mastery = jax.nn.sigmoid(_mlp(params["head_mast"], xc))[:, 0]
    return ({"student": xs, "question": xq, "concept": xc}, logits, mastery)


# speedup vs baseline: 21.9005x; 1.0153x over previous
import functools

import jax
import jax.numpy as jnp
from jax import lax
from jax.experimental import pallas as pl
from jax.experimental.pallas import tpu as pltpu
from jax.experimental.pallas import tpu_sc as plsc

NCORE = 2
NSUB = 16
LANE = 16
B = 160
H = 32
AUGW = 48


def _cdiv(a, b):
    return (a + b - 1) // b


@functools.lru_cache(maxsize=None)
def _edge_kernel(e_pad, e_real, n_src, half, split):
    P = _cdiv(half, 2048) * 2048
    c16 = P // 16
    assert c16 % B == 0
    zcnt = c16 // B
    nblk = (e_pad // (NSUB if split else NSUB * NCORE)) // B
    mesh = plsc.VectorSubcoreMesh(core_axis_name="c", subcore_axis_name="s")

    def body(aug_hbm, ad_hbm, src_hbm, dst_hbm, acc_hbm, den_hbm,
             acc_sh, den_sh, sidx, didx, rows, msg, adv, exv, dstl, sem):
        c = lax.axis_index("c")
        s = lax.axis_index("s")
        lo = c * half if split else 0
        lanes = lax.iota(jnp.int32, 16)
        col_as = jnp.full((16,), H, jnp.int32)
        zv = jnp.zeros((16,), jnp.float32)

        def z2(i, _):
            msg[i // 2, pl.ds((i % 2) * 16, 16)] = zv
            return _
        lax.fori_loop(0, B * 2, z2, None)

        def z1(i, _):
            exv[pl.ds(i * 16, 16)] = zv
            return _
        lax.fori_loop(0, B // 16, z1, None)

        for k in range(zcnt):
            r0 = pl.multiple_of(s * c16 + k * B, 8)
            pltpu.sync_copy(msg, acc_sh.at[pl.ds(r0, B)])
            pltpu.sync_copy(exv, den_sh.at[pl.ds(r0, B)])
        plsc.subcore_barrier()

        base = (s * (e_pad // NSUB)) if split else (
            (s * NCORE + c) * (e_pad // (NSUB * NCORE)))

        def blk(b, _):
            e0 = pl.multiple_of(base + b * B, B)
            pltpu.sync_copy(src_hbm.at[pl.ds(e0, B)], sidx)
            pltpu.sync_copy(dst_hbm.at[pl.ds(e0, B)], didx)
            pltpu.async_copy(aug_hbm.at[sidx], rows, sem).wait()
            pltpu.async_copy(ad_hbm.at[didx], adv, sem).wait()

            def chunk(j, _):
                d16 = didx[pl.ds(j * 16, 16)]
                gidx = e0 + j * 16 + lanes
                valid = (gidx < e_real) & (d16 >= lo) & (d16 < lo + half)
                dl = jnp.where(valid, d16 - lo, 0)
                ad16 = adv[pl.ds(j * 16, 16)]
                as16 = plsc.load_gather(rows, [j * 16 + lanes, col_as])
                z = as16 + ad16
                e = jnp.maximum(z, 0.2 * z)
                ex = jnp.where(valid, jnp.exp(e), 0.0)
                exv[pl.ds(j * 16, 16)] = ex
                dstl[pl.ds(j * 16, 16)] = dl
                return _
            lax.fori_loop(0, B // 16, chunk, None)

            def scale(i, _):
                i16 = jnp.full((16,), i, jnp.int32)
                exb = plsc.load_gather(exv, [i16])
                r0 = plsc.load_gather(rows, [i16, lanes])
                r1 = plsc.load_gather(rows, [i16, lanes + 16])
                plsc.store_scatter(msg, [i16, lanes], r0 * exb)
                plsc.store_scatter(msg, [i16, lanes + 16], r1 * exb)
                return _
            lax.fori_loop(0, B, scale, None)

            pltpu.sync_copy(msg, acc_sh.at[dstl], add=True)
            pltpu.sync_copy(exv, den_sh.at[dstl], add=True)
            return _
        lax.fori_loop(0, nblk, blk, None)

        plsc.subcore_barrier()
        for k in range(zcnt):
            r0 = pl.multiple_of(s * c16 + k * B, 8)
            pltpu.sync_copy(acc_sh.at[pl.ds(r0, B)],
                            acc_hbm.at[c, pl.ds(r0, B)])
            d0 = pl.multiple_of(c * P + s * c16 + k * B, 8)
            pltpu.sync_copy(den_sh.at[pl.ds(r0, B)], den_hbm.at[pl.ds(d0, B)])

    return pl.kernel(
        body,
        mesh=mesh,
        compiler_params=pltpu.CompilerParams(
            needs_layout_passes=False, use_tc_tiling_on_sc=False),
        out_type=(jax.ShapeDtypeStruct((NCORE, P, H), jnp.float32),
                  jax.ShapeDtypeStruct((NCORE * P,), jnp.float32)),
        scratch_types=[
            pltpu.VMEM_SHARED((P, H), jnp.float32),
            pltpu.VMEM_SHARED((P,), jnp.float32),
            pltpu.VMEM((B,), jnp.int32),
            pltpu.VMEM((B,), jnp.int32),
            pltpu.VMEM((B, AUGW), jnp.float32),
            pltpu.VMEM((B, H), jnp.float32),
            pltpu.VMEM((B,), jnp.float32),
            pltpu.VMEM((B,), jnp.float32),
            pltpu.VMEM((B,), jnp.int32),
            pltpu.SemaphoreType.DMA,
        ],
    )


@functools.lru_cache(maxsize=None)
def _gather_kernel(l_pad, n_rows):
    nblk = l_pad // (NSUB * NCORE * B)
    mesh = plsc.VectorSubcoreMesh(core_axis_name="c", subcore_axis_name="s")

    def body(tab_hbm, idx_hbm, out_hbm, idxv, rowsv, sem):
        c = lax.axis_index("c")
        s = lax.axis_index("s")
        base = (s * NCORE + c) * (l_pad // (NSUB * NCORE))

        def blk(b, _):
            e0 = pl.multiple_of(base + b * B, B)
            pltpu.sync_copy(idx_hbm.at[pl.ds(e0, B)], idxv)
            pltpu.async_copy(tab_hbm.at[idxv], rowsv, sem).wait()
            pltpu.sync_copy(rowsv, out_hbm.at[pl.ds(e0, B)])
            return _
        lax.fori_loop(0, nblk, blk, None)

    return pl.kernel(
        body,
        mesh=mesh,
        compiler_params=pltpu.CompilerParams(
            needs_layout_passes=False, use_tc_tiling_on_sc=False),
        out_type=jax.ShapeDtypeStruct((l_pad, H), jnp.float32),
        scratch_types=[
            pltpu.VMEM((B,), jnp.int32),
            pltpu.VMEM((B, H), jnp.float32),
            pltpu.SemaphoreType.DMA,
        ],
    )


def _gather_rows(table, idx):
    l = idx.shape[0]
    l_pad = _cdiv(l, NSUB * NCORE * B) * (NSUB * NCORE * B)
    if l_pad != l:
        idx = jnp.concatenate([idx, jnp.zeros((l_pad - l,), idx.dtype)])
    return _gather_kernel(l_pad, table.shape[0])(table, idx)[:l]


def _rb(n):
    return 2000 if n % 2000 == 0 else 2048


def _full(shape):
    return pl.BlockSpec(shape, lambda i: tuple(0 for _ in shape))


def _enc_gru_tc(x, h0, pe, pg):
    n, br = x.shape[0], _rb(x.shape[0])

    def body(x_r, h_r, w_r, b_r, wih_r, whh_r, bih_r, bhh_r, o_r):
        xb = jnp.dot(x_r[...], w_r[...],
                     preferred_element_type=jnp.float32) + b_r[...]
        gi = jnp.dot(xb, wih_r[...],
                     preferred_element_type=jnp.float32) + bih_r[...]
        gh = jnp.dot(h_r[...], whh_r[...],
                     preferred_element_type=jnp.float32) + bhh_r[...]
        ir, iz, inn = jnp.split(gi, 3, axis=1)
        hr, hz, hn = jnp.split(gh, 3, axis=1)
        r = jax.nn.sigmoid(ir + hr)
        z = jax.nn.sigmoid(iz + hz)
        nn = jnp.tanh(inn + r * hn)
        o_r[...] = (1.0 - z) * nn + z * h_r[...]

    fi = x.shape[1]
    return pl.pallas_call(
        body,
        grid=(n // br,),
        in_specs=[pl.BlockSpec((br, fi), lambda i: (i, 0)),
                  pl.BlockSpec((br, H), lambda i: (i, 0)),
                  _full((fi, H)), _full((H,)),
                  _full((H, 3 * H)), _full((H, 3 * H)),
                  _full((3 * H,)), _full((3 * H,))],
        out_specs=pl.BlockSpec((br, H), lambda i: (i, 0)),
        out_shape=jax.ShapeDtypeStruct((n, H), jnp.float32),
    )(x, h0, pe["W"], pe["b"], pg["W_ih"], pg["W_hh"], pg["b_ih"], pg["b_hh"])


def _enc_tc(x, p):
    n, br, fi = x.shape[0], _rb(x.shape[0]), x.shape[1]

    def body(x_r, w_r, b_r, o_r):
        o_r[...] = jnp.dot(x_r[...], w_r[...],
                           preferred_element_type=jnp.float32) + b_r[...]

    return pl.pallas_call(
        body,
        grid=(n // br,),
        in_specs=[pl.BlockSpec((br, fi), lambda i: (i, 0)),
                  _full((fi, H)), _full((H,))],
        out_specs=pl.BlockSpec((br, H), lambda i: (i, 0)),
        out_shape=jax.ShapeDtypeStruct((n, H), jnp.float32),
    )(x, p["W"], p["b"])


def _aug_tc(x, w_src, a_src):
    n, br = x.shape[0], _rb(x.shape[0])

    def body(x_r, w_r, a_r, o_r):
        hs = jnp.dot(x_r[...], w_r[...], preferred_element_type=jnp.float32)
        a_s = jnp.dot(hs, a_r[...][:, None],
                      preferred_element_type=jnp.float32)
        o_r[...] = jnp.concatenate(
            [hs, a_s, jnp.zeros((br, AUGW - H - 1), jnp.float32)], axis=1)

    return pl.pallas_call(
        body,
        grid=(n // br,),
        in_specs=[pl.BlockSpec((br, H), lambda i: (i, 0)),
                  _full((H, H)), _full((H,))],
        out_specs=pl.BlockSpec((br, AUGW), lambda i: (i, 0)),
        out_shape=jax.ShapeDtypeStruct((n, AUGW), jnp.float32),
    )(x, w_src, a_src)


def _advec_tc(x, wd, av):
    n, br, r = x.shape[0], _rb(x.shape[0]), wd.shape[0]

    def body(x_r, wd_r, av_r, o_r):
        wds, avs = wd_r[...], av_r[...]
        cols = jnp.stack(
            [jnp.dot(wds[j], avs[j], preferred_element_type=jnp.float32)
             for j in range(r)], axis=1)
        o_r[...] = jnp.dot(x_r[...], cols,
                           preferred_element_type=jnp.float32)

    return pl.pallas_call(
        body,
        grid=(n // br,),
        in_specs=[pl.BlockSpec((br, H), lambda i: (i, 0)),
                  _full((r, H, H)), _full((r, H))],
        out_specs=pl.BlockSpec((br, r), lambda i: (i, 0)),
        out_shape=jax.ShapeDtypeStruct((n, r), jnp.float32),
    )(x, wd, av)


def _epilogue_tc(accs, dens, biases):
    r, n = accs[0].shape[0], accs[0].shape[1]
    br = _rb(n)
    ncopy = len(accs)
    dens = [d[..., None] for d in dens]

    def body(*refs):
        o_r = refs[-1]
        acc_v = [rf[...] for rf in refs[:ncopy]]
        den_v = [rf[...] for rf in refs[ncopy:2 * ncopy]]
        b_v = refs[2 * ncopy][...]
        out = jnp.zeros((br, H), jnp.float32)
        for j in range(r):
            a = acc_v[0][j]
            d = den_v[0][j]
            for k in range(1, ncopy):
                a = a + acc_v[k][j]
                d = d + den_v[k][j]
            out = out + a / (d + 1e-16) + b_v[j]
        o_r[...] = jax.nn.relu(out)

    return pl.pallas_call(
        body,
        grid=(n // br,),
        in_specs=([pl.BlockSpec((r, br, H), lambda i: (0, i, 0))] * ncopy
                  + [pl.BlockSpec((r, br, 1), lambda i: (0, i, 0))] * ncopy
                  + [_full((r, H))]),
        out_specs=pl.BlockSpec((br, H), lambda i: (i, 0)),
        out_shape=jax.ShapeDtypeStruct((n, H), jnp.float32),
    )(*accs, *dens, biases)


def _corr_head_tc(ag, bg, p):
    n, br = ag.shape[0], _rb(ag.shape[0])
    w1a, w1b = p["l1"]["W"][:H], p["l1"]["W"][H:]

    def body(a_r, b_r, wa_r, wb_r, b1_r, w2_r, b2_r, o_r):
        h = jax.nn.relu(
            jnp.dot(a_r[...], wa_r[...], preferred_element_type=jnp.float32)
            + jnp.dot(b_r[...], wb_r[...], preferred_element_type=jnp.float32)
            + b1_r[...])
        o_r[...] = jnp.dot(h, w2_r[...],
                           preferred_element_type=jnp.float32) + b2_r[...]

    return pl.pallas_call(
        body,
        grid=(n // br,),
        in_specs=[pl.BlockSpec((br, H), lambda i: (i, 0)),
                  pl.BlockSpec((br, H), lambda i: (i, 0)),
                  _full((H, H)), _full((H, H)), _full((H,)),
                  _full((H, 1)), _full((1,))],
        out_specs=pl.BlockSpec((br, 1), lambda i: (i, 0)),
        out_shape=jax.ShapeDtypeStruct((n, 1), jnp.float32),
    )(ag, bg, w1a, w1b, p["l1"]["b"], p["l2"]["W"], p["l2"]["b"])[:, 0]


def _mast_head_tc(xc, p):
    n, br = xc.shape[0], _rb(xc.shape[0])

    def body(x_r, w1_r, b1_r, w2_r, b2_r, o_r):
        h = jax.nn.relu(jnp.dot(x_r[...], w1_r[...],
                                preferred_element_type=jnp.float32) + b1_r[...])
        o_r[...] = jax.nn.sigmoid(
            jnp.dot(h, w2_r[...], preferred_element_type=jnp.float32)
            + b2_r[...])

    return pl.pallas_call(
        body,
        grid=(n // br,),
        in_specs=[pl.BlockSpec((br, H), lambda i: (i, 0)),
                  _full((H, H)), _full((H,)), _full((H, 1)), _full((1,))],
        out_specs=pl.BlockSpec((br, 1), lambda i: (i, 0)),
        out_shape=jax.ShapeDtypeStruct((n, 1), jnp.float32),
    )(xc, p["l1"]["W"], p["l1"]["b"], p["l2"]["W"], p["l2"]["b"])[:, 0]


def _pad_edges(src, dst):
    e = src.shape[0]
    e_pad = _cdiv(e, NSUB * NCORE * B) * (NSUB * NCORE * B)
    if e_pad != e:
        pad = e_pad - e
        src = jnp.concatenate([src, jnp.zeros((pad,), src.dtype)])
        dst = jnp.concatenate([dst, jnp.zeros((pad,), dst.dtype)])
    return src, dst, e


def _gat_sc(aug, ad, src_p, dst_p, e_real, n_dst):
    split = n_dst * H * 4 > 7 * 2**20
    half = n_dst // NCORE if split else n_dst
    P = _cdiv(half, 2048) * 2048
    k = _edge_kernel(src_p.shape[0], e_real, aug.shape[0], half, split)
    acc, den = k(aug, ad, src_p, dst_p)
    den = den.reshape(NCORE, P)
    if split:
        return ([acc[:, :half].reshape(n_dst, H)],
                [den[:, :half].reshape(n_dst)])
    return ([acc[0][:half], acc[1][:half]], [den[0][:half], den[1][:half]])


def _node_out(gats, biases):
    ncopy = len(gats[0][0])
    accs = [jnp.stack([g[0][k] for g in gats]) for k in range(ncopy)]
    dens = [jnp.stack([g[1][k] for g in gats]) for k in range(ncopy)]
    return _epilogue_tc(accs, dens, jnp.stack(biases))


def kernel(x_student, x_question, x_concept, prev_h, e_att_src, e_att_dst,
           e_tag_src, e_tag_dst, e_pre_src, e_pre_dst, e_mas_src, e_mas_dst,
           label_src, label_dst, params):
    ns, nq, nc = x_student.shape[0], x_question.shape[0], x_concept.shape[0]
    att_s, att_d, att_e = _pad_edges(e_att_src, e_att_dst)
    tag_s, tag_d, tag_e = _pad_edges(e_tag_src, e_tag_dst)
    pre_s, pre_d, pre_e = _pad_edges(e_pre_src, e_pre_dst)
    mas_s, mas_d, mas_e = _pad_edges(e_mas_src, e_mas_dst)

    xs = _enc_gru_tc(x_student, prev_h, params["enc_s"], params["gru"])
    xq = _enc_tc(x_question, params["enc_q"])
    xc = _enc_tc(x_concept, params["enc_c"])

    def hetero(conv, xs, xq, xc):
        def aug(r, x):
            return _aug_tc(x, conv[r]["W_src"], conv[r]["a_src"])

        def advec(x, rels):
            return _advec_tc(x, jnp.stack([conv[r]["W_dst"] for r in rels]),
                             jnp.stack([conv[r]["a_dst"] for r in rels]))

        ad_s = advec(xs, ["rev_att", "rev_mas"])
        ad_q = advec(xq, ["att", "rev_tag"])
        ad_c = advec(xc, ["tag", "pre", "mas"])
        s = _node_out(
            [_gat_sc(aug("rev_att", xq), ad_s[:, 0], att_d, att_s, att_e, ns),
             _gat_sc(aug("rev_mas", xc), ad_s[:, 1], mas_d, mas_s, mas_e, ns)],
            [conv["rev_att"]["bias"], conv["rev_mas"]["bias"]])
        q = _node_out(
            [_gat_sc(aug("att", xs), ad_q[:, 0], att_s, att_d, att_e, nq),
             _gat_sc(aug("rev_tag", xc), ad_q[:, 1], tag_d, tag_s, tag_e, nq)],
            [conv["att"]["bias"], conv["rev_tag"]["bias"]])
        c = _node_out(
            [_gat_sc(aug("tag", xq), ad_c[:, 0], tag_s, tag_d, tag_e, nc),
             _gat_sc(aug("pre", xc), ad_c[:, 1], pre_s, pre_d, pre_e, nc),
             _gat_sc(aug("mas", xs), ad_c[:, 2], mas_s, mas_d, mas_e, nc)],
            [conv["tag"]["bias"], conv["pre"]["bias"], conv["mas"]["bias"]])
        return s, q, c

    xs, xq, xc = hetero(params["conv1"], xs, xq, xc)
    xs, xq, xc = hetero(params["conv2"], xs, xq, xc)

    ag = _gather_rows(xs, label_src)
    bg = _gather_rows(xq, label_dst)
    logits = _corr_head_tc(ag, bg, params["head_corr"])
    mastery = _mast_head_tc(xc, params["head_mast"])
    return ({"student": xs, "question": xq, "concept": xc}, logits, mastery)


# speedup vs baseline: 25.7801x; 1.1771x over previous
import functools

import jax
import jax.numpy as jnp
from jax import lax
from jax.experimental import pallas as pl
from jax.experimental.pallas import tpu as pltpu
from jax.experimental.pallas import tpu_sc as plsc

NCORE = 2
NSUB = 16
LANE = 16
B = 160
H = 32
AUGW = 48


def _cdiv(a, b):
    return (a + b - 1) // b


@functools.lru_cache(maxsize=None)
def _edge_kernel(e_pad, e_real, n_src, half, split):
    P = _cdiv(half, 2048) * 2048
    c16 = P // 16
    assert c16 % B == 0
    zcnt = c16 // B
    nblk = (e_pad // (NSUB if split else NSUB * NCORE)) // B
    mesh = plsc.VectorSubcoreMesh(core_axis_name="c", subcore_axis_name="s")

    def body(aug_hbm, ad_hbm, src_hbm, dst_hbm, acc_hbm, den_hbm,
             acc_sh, den_sh, sidx, didx, rows, msg, adv, exv, dstl, sem, sem2):
        c = lax.axis_index("c")
        s = lax.axis_index("s")
        lo = c * half if split else 0
        lanes = lax.iota(jnp.int32, 16)
        col_as = jnp.full((16,), H, jnp.int32)
        zv = jnp.zeros((16,), jnp.float32)

        def z2(i, _):
            msg[i // 2, pl.ds((i % 2) * 16, 16)] = zv
            return _
        lax.fori_loop(0, B * 2, z2, None)

        def z1(i, _):
            exv[pl.ds(i * 16, 16)] = zv
            return _
        lax.fori_loop(0, B // 16, z1, None)

        for k in range(zcnt):
            r0 = pl.multiple_of(s * c16 + k * B, 8)
            pltpu.sync_copy(msg, acc_sh.at[pl.ds(r0, B)])
            pltpu.sync_copy(exv, den_sh.at[pl.ds(r0, B)])
        plsc.subcore_barrier()

        base = (s * (e_pad // NSUB)) if split else (
            (s * NCORE + c) * (e_pad // (NSUB * NCORE)))

        def blk(b, _):
            e0 = pl.multiple_of(base + b * B, B)
            pltpu.sync_copy(src_hbm.at[pl.ds(e0, B)], sidx)
            pltpu.sync_copy(dst_hbm.at[pl.ds(e0, B)], didx)
            h1 = pltpu.async_copy(aug_hbm.at[sidx], rows, sem)
            h2 = pltpu.async_copy(ad_hbm.at[didx], adv, sem2)
            h1.wait()
            h2.wait()

            def chunk(j, _):
                d16 = didx[pl.ds(j * 16, 16)]
                gidx = e0 + j * 16 + lanes
                valid = (gidx < e_real) & (d16 >= lo) & (d16 < lo + half)
                dl = jnp.where(valid, d16 - lo, 0)
                ad16 = adv[pl.ds(j * 16, 16)]
                as16 = plsc.load_gather(rows, [j * 16 + lanes, col_as])
                z = as16 + ad16
                e = jnp.maximum(z, 0.2 * z)
                ex = jnp.where(valid, jnp.exp(e), 0.0)
                exv[pl.ds(j * 16, 16)] = ex
                dstl[pl.ds(j * 16, 16)] = dl
                return _
            lax.fori_loop(0, B // 16, chunk, None)

            def scale(i, _):
                i16 = jnp.full((16,), i, jnp.int32)
                exb = plsc.load_gather(exv, [i16])
                r0 = plsc.load_gather(rows, [i16, lanes])
                r1 = plsc.load_gather(rows, [i16, lanes + 16])
                plsc.store_scatter(msg, [i16, lanes], r0 * exb)
                plsc.store_scatter(msg, [i16, lanes + 16], r1 * exb)
                return _
            lax.fori_loop(0, B, scale, None)

            pltpu.sync_copy(msg, acc_sh.at[dstl], add=True)
            pltpu.sync_copy(exv, den_sh.at[dstl], add=True)
            return _
        lax.fori_loop(0, nblk, blk, None)

        plsc.subcore_barrier()
        for k in range(zcnt):
            r0 = pl.multiple_of(s * c16 + k * B, 8)
            pltpu.sync_copy(acc_sh.at[pl.ds(r0, B)],
                            acc_hbm.at[c, pl.ds(r0, B)])
            d0 = pl.multiple_of(c * P + s * c16 + k * B, 8)
            pltpu.sync_copy(den_sh.at[pl.ds(r0, B)], den_hbm.at[pl.ds(d0, B)])

    return pl.kernel(
        body,
        mesh=mesh,
        compiler_params=pltpu.CompilerParams(
            needs_layout_passes=False, use_tc_tiling_on_sc=False),
        out_type=(jax.ShapeDtypeStruct((NCORE, P, H), jnp.float32),
                  jax.ShapeDtypeStruct((NCORE * P,), jnp.float32)),
        scratch_types=[
            pltpu.VMEM_SHARED((P, H), jnp.float32),
            pltpu.VMEM_SHARED((P,), jnp.float32),
            pltpu.VMEM((B,), jnp.int32),
            pltpu.VMEM((B,), jnp.int32),
            pltpu.VMEM((B, AUGW), jnp.float32),
            pltpu.VMEM((B, H), jnp.float32),
            pltpu.VMEM((B,), jnp.float32),
            pltpu.VMEM((B,), jnp.float32),
            pltpu.VMEM((B,), jnp.int32),
            pltpu.SemaphoreType.DMA,
            pltpu.SemaphoreType.DMA,
        ],
    )


@functools.lru_cache(maxsize=None)
def _gather_kernel(l_pad, n_rows):
    nblk = l_pad // (NSUB * NCORE * B)
    mesh = plsc.VectorSubcoreMesh(core_axis_name="c", subcore_axis_name="s")

    def body(tab_hbm, idx_hbm, out_hbm, idxv, rowsv, sem):
        c = lax.axis_index("c")
        s = lax.axis_index("s")
        base = (s * NCORE + c) * (l_pad // (NSUB * NCORE))

        def blk(b, _):
            e0 = pl.multiple_of(base + b * B, B)
            pltpu.sync_copy(idx_hbm.at[pl.ds(e0, B)], idxv)
            pltpu.async_copy(tab_hbm.at[idxv], rowsv, sem).wait()
            pltpu.sync_copy(rowsv, out_hbm.at[pl.ds(e0, B)])
            return _
        lax.fori_loop(0, nblk, blk, None)

    return pl.kernel(
        body,
        mesh=mesh,
        compiler_params=pltpu.CompilerParams(
            needs_layout_passes=False, use_tc_tiling_on_sc=False),
        out_type=jax.ShapeDtypeStruct((l_pad, H), jnp.float32),
        scratch_types=[
            pltpu.VMEM((B,), jnp.int32),
            pltpu.VMEM((B, H), jnp.float32),
            pltpu.SemaphoreType.DMA,
        ],
    )


def _gather_rows(table, idx):
    l = idx.shape[0]
    l_pad = _cdiv(l, NSUB * NCORE * B) * (NSUB * NCORE * B)
    if l_pad != l:
        idx = jnp.concatenate([idx, jnp.zeros((l_pad - l,), idx.dtype)])
    return _gather_kernel(l_pad, table.shape[0])(table, idx)[:l]


def _rb(n):
    return 2000 if n % 2000 == 0 else 2048


def _full(shape):
    return pl.BlockSpec(shape, lambda i: tuple(0 for _ in shape))


def _enc_gru_tc(x, h0, pe, pg):
    n, br = x.shape[0], _rb(x.shape[0])

    def body(x_r, h_r, w_r, b_r, wih_r, whh_r, bih_r, bhh_r, o_r):
        xb = jnp.dot(x_r[...], w_r[...],
                     preferred_element_type=jnp.float32) + b_r[...]
        gi = jnp.dot(xb, wih_r[...],
                     preferred_element_type=jnp.float32) + bih_r[...]
        gh = jnp.dot(h_r[...], whh_r[...],
                     preferred_element_type=jnp.float32) + bhh_r[...]
        ir, iz, inn = jnp.split(gi, 3, axis=1)
        hr, hz, hn = jnp.split(gh, 3, axis=1)
        r = jax.nn.sigmoid(ir + hr)
        z = jax.nn.sigmoid(iz + hz)
        nn = jnp.tanh(inn + r * hn)
        o_r[...] = (1.0 - z) * nn + z * h_r[...]

    fi = x.shape[1]
    return pl.pallas_call(
        body,
        grid=(n // br,),
        in_specs=[pl.BlockSpec((br, fi), lambda i: (i, 0)),
                  pl.BlockSpec((br, H), lambda i: (i, 0)),
                  _full((fi, H)), _full((H,)),
                  _full((H, 3 * H)), _full((H, 3 * H)),
                  _full((3 * H,)), _full((3 * H,))],
        out_specs=pl.BlockSpec((br, H), lambda i: (i, 0)),
        out_shape=jax.ShapeDtypeStruct((n, H), jnp.float32),
    )(x, h0, pe["W"], pe["b"], pg["W_ih"], pg["W_hh"], pg["b_ih"], pg["b_hh"])


def _enc_tc(x, p):
    n, br, fi = x.shape[0], _rb(x.shape[0]), x.shape[1]

    def body(x_r, w_r, b_r, o_r):
        o_r[...] = jnp.dot(x_r[...], w_r[...],
                           preferred_element_type=jnp.float32) + b_r[...]

    return pl.pallas_call(
        body,
        grid=(n // br,),
        in_specs=[pl.BlockSpec((br, fi), lambda i: (i, 0)),
                  _full((fi, H)), _full((H,))],
        out_specs=pl.BlockSpec((br, H), lambda i: (i, 0)),
        out_shape=jax.ShapeDtypeStruct((n, H), jnp.float32),
    )(x, p["W"], p["b"])


def _aug_tc(x, w_src, a_src):
    n, br = x.shape[0], _rb(x.shape[0])

    def body(x_r, w_r, a_r, o_r):
        hs = jnp.dot(x_r[...], w_r[...], preferred_element_type=jnp.float32)
        a_s = jnp.dot(hs, a_r[...][:, None],
                      preferred_element_type=jnp.float32)
        o_r[...] = jnp.concatenate(
            [hs, a_s, jnp.zeros((br, AUGW - H - 1), jnp.float32)], axis=1)

    return pl.pallas_call(
        body,
        grid=(n // br,),
        in_specs=[pl.BlockSpec((br, H), lambda i: (i, 0)),
                  _full((H, H)), _full((H,))],
        out_specs=pl.BlockSpec((br, AUGW), lambda i: (i, 0)),
        out_shape=jax.ShapeDtypeStruct((n, AUGW), jnp.float32),
    )(x, w_src, a_src)


def _advec_tc(x, wd, av):
    n, br, r = x.shape[0], _rb(x.shape[0]), wd.shape[0]

    def body(x_r, wd_r, av_r, o_r):
        wds, avs = wd_r[...], av_r[...]
        cols = jnp.stack(
            [jnp.dot(wds[j], avs[j], preferred_element_type=jnp.float32)
             for j in range(r)], axis=1)
        o_r[...] = jnp.dot(x_r[...], cols,
                           preferred_element_type=jnp.float32)

    return pl.pallas_call(
        body,
        grid=(n // br,),
        in_specs=[pl.BlockSpec((br, H), lambda i: (i, 0)),
                  _full((r, H, H)), _full((r, H))],
        out_specs=pl.BlockSpec((br, r), lambda i: (i, 0)),
        out_shape=jax.ShapeDtypeStruct((n, r), jnp.float32),
    )(x, wd, av)


def _epilogue_tc(accs, dens, biases):
    r, n = accs[0].shape[0], accs[0].shape[1]
    br = _rb(n)
    ncopy = len(accs)
    dens = [d[..., None] for d in dens]

    def body(*refs):
        o_r = refs[-1]
        acc_v = [rf[...] for rf in refs[:ncopy]]
        den_v = [rf[...] for rf in refs[ncopy:2 * ncopy]]
        b_v = refs[2 * ncopy][...]
        out = jnp.zeros((br, H), jnp.float32)
        for j in range(r):
            a = acc_v[0][j]
            d = den_v[0][j]
            for k in range(1, ncopy):
                a = a + acc_v[k][j]
                d = d + den_v[k][j]
            out = out + a / (d + 1e-16) + b_v[j]
        o_r[...] = jax.nn.relu(out)

    return pl.pallas_call(
        body,
        grid=(n // br,),
        in_specs=([pl.BlockSpec((r, br, H), lambda i: (0, i, 0))] * ncopy
                  + [pl.BlockSpec((r, br, 1), lambda i: (0, i, 0))] * ncopy
                  + [_full((r, H))]),
        out_specs=pl.BlockSpec((br, H), lambda i: (i, 0)),
        out_shape=jax.ShapeDtypeStruct((n, H), jnp.float32),
    )(*accs, *dens, biases)


def _corr_head_tc(ag, bg, p):
    n, br = ag.shape[0], _rb(ag.shape[0])
    w1a, w1b = p["l1"]["W"][:H], p["l1"]["W"][H:]

    def body(a_r, b_r, wa_r, wb_r, b1_r, w2_r, b2_r, o_r):
        h = jax.nn.relu(
            jnp.dot(a_r[...], wa_r[...], preferred_element_type=jnp.float32)
            + jnp.dot(b_r[...], wb_r[...], preferred_element_type=jnp.float32)
            + b1_r[...])
        o_r[...] = jnp.dot(h, w2_r[...],
                           preferred_element_type=jnp.float32) + b2_r[...]

    return pl.pallas_call(
        body,
        grid=(n // br,),
        in_specs=[pl.BlockSpec((br, H), lambda i: (i, 0)),
                  pl.BlockSpec((br, H), lambda i: (i, 0)),
                  _full((H, H)), _full((H, H)), _full((H,)),
                  _full((H, 1)), _full((1,))],
        out_specs=pl.BlockSpec((br, 1), lambda i: (i, 0)),
        out_shape=jax.ShapeDtypeStruct((n, 1), jnp.float32),
    )(ag, bg, w1a, w1b, p["l1"]["b"], p["l2"]["W"], p["l2"]["b"])[:, 0]


def _mast_head_tc(xc, p):
    n, br = xc.shape[0], _rb(xc.shape[0])

    def body(x_r, w1_r, b1_r, w2_r, b2_r, o_r):
        h = jax.nn.relu(jnp.dot(x_r[...], w1_r[...],
                                preferred_element_type=jnp.float32) + b1_r[...])
        o_r[...] = jax.nn.sigmoid(
            jnp.dot(h, w2_r[...], preferred_element_type=jnp.float32)
            + b2_r[...])

    return pl.pallas_call(
        body,
        grid=(n // br,),
        in_specs=[pl.BlockSpec((br, H), lambda i: (i, 0)),
                  _full((H, H)), _full((H,)), _full((H, 1)), _full((1,))],
        out_specs=pl.BlockSpec((br, 1), lambda i: (i, 0)),
        out_shape=jax.ShapeDtypeStruct((n, 1), jnp.float32),
    )(xc, p["l1"]["W"], p["l1"]["b"], p["l2"]["W"], p["l2"]["b"])[:, 0]


def _pad_edges(src, dst):
    e = src.shape[0]
    e_pad = _cdiv(e, NSUB * NCORE * B) * (NSUB * NCORE * B)
    if e_pad != e:
        pad = e_pad - e
        src = jnp.concatenate([src, jnp.zeros((pad,), src.dtype)])
        dst = jnp.concatenate([dst, jnp.zeros((pad,), dst.dtype)])
    return src, dst, e


def _gat_sc(aug, ad, src_p, dst_p, e_real, n_dst):
    split = n_dst * H * 4 > 7 * 2**20
    half = n_dst // NCORE if split else n_dst
    P = _cdiv(half, 2048) * 2048
    k = _edge_kernel(src_p.shape[0], e_real, aug.shape[0], half, split)
    acc, den = k(aug, ad, src_p, dst_p)
    den = den.reshape(NCORE, P)
    if split:
        return ([acc[:, :half].reshape(n_dst, H)],
                [den[:, :half].reshape(n_dst)])
    return ([acc[0][:half], acc[1][:half]], [den[0][:half], den[1][:half]])


def _node_out(gats, biases):
    ncopy = len(gats[0][0])
    accs = [jnp.stack([g[0][k] for g in gats]) for k in range(ncopy)]
    dens = [jnp.stack([g[1][k] for g in gats]) for k in range(ncopy)]
    return _epilogue_tc(accs, dens, jnp.stack(biases))


def kernel(x_student, x_question, x_concept, prev_h, e_att_src, e_att_dst,
           e_tag_src, e_tag_dst, e_pre_src, e_pre_dst, e_mas_src, e_mas_dst,
           label_src, label_dst, params):
    ns, nq, nc = x_student.shape[0], x_question.shape[0], x_concept.shape[0]
    att_s, att_d, att_e = _pad_edges(e_att_src, e_att_dst)
    tag_s, tag_d, tag_e = _pad_edges(e_tag_src, e_tag_dst)
    pre_s, pre_d, pre_e = _pad_edges(e_pre_src, e_pre_dst)
    mas_s, mas_d, mas_e = _pad_edges(e_mas_src, e_mas_dst)

    xs = _enc_gru_tc(x_student, prev_h, params["enc_s"], params["gru"])
    xq = _enc_tc(x_question, params["enc_q"])
    xc = _enc_tc(x_concept, params["enc_c"])

    def hetero(conv, xs, xq, xc):
        def aug(r, x):
            return _aug_tc(x, conv[r]["W_src"], conv[r]["a_src"])

        def advec(x, rels):
            return _advec_tc(x, jnp.stack([conv[r]["W_dst"] for r in rels]),
                             jnp.stack([conv[r]["a_dst"] for r in rels]))

        ad_s = advec(xs, ["rev_att", "rev_mas"])
        ad_q = advec(xq, ["att", "rev_tag"])
        ad_c = advec(xc, ["tag", "pre", "mas"])
        s = _node_out(
            [_gat_sc(aug("rev_att", xq), ad_s[:, 0], att_d, att_s, att_e, ns),
             _gat_sc(aug("rev_mas", xc), ad_s[:, 1], mas_d, mas_s, mas_e, ns)],
            [conv["rev_att"]["bias"], conv["rev_mas"]["bias"]])
        q = _node_out(
            [_gat_sc(aug("att", xs), ad_q[:, 0], att_s, att_d, att_e, nq),
             _gat_sc(aug("rev_tag", xc), ad_q[:, 1], tag_d, tag_s, tag_e, nq)],
            [conv["att"]["bias"], conv["rev_tag"]["bias"]])
        c = _node_out(
            [_gat_sc(aug("tag", xq), ad_c[:, 0], tag_s, tag_d, tag_e, nc),
             _gat_sc(aug("pre", xc), ad_c[:, 1], pre_s, pre_d, pre_e, nc),
             _gat_sc(aug("mas", xs), ad_c[:, 2], mas_s, mas_d, mas_e, nc)],
            [conv["tag"]["bias"], conv["pre"]["bias"], conv["mas"]["bias"]])
        return s, q, c

    xs, xq, xc = hetero(params["conv1"], xs, xq, xc)
    xs, xq, xc = hetero(params["conv2"], xs, xq, xc)

    ag = _gather_rows(xs, label_src)
    bg = _gather_rows(xq, label_dst)
    logits = _corr_head_tc(ag, bg, params["head_corr"])
    mastery = _mast_head_tc(xc, params["head_mast"])
    return ({"student": xs, "question": xq, "concept": xc}, logits, mastery)


# speedup vs baseline: 28.3469x; 1.0996x over previous
import functools

import jax
import jax.numpy as jnp
from jax import lax
from jax.experimental import pallas as pl
from jax.experimental.pallas import tpu as pltpu
from jax.experimental.pallas import tpu_sc as plsc

NCORE = 2
NSUB = 16
LANE = 16
B = 160
H = 32
AUGW = 48


def _cdiv(a, b):
    return (a + b - 1) // b


@functools.lru_cache(maxsize=None)
def _edge_kernel(e_pad, e_real, n_src, half, split):
    P = _cdiv(half, 2048) * 2048
    c16 = P // 16
    assert c16 % B == 0
    zcnt = c16 // B
    nblk = (e_pad // (NSUB if split else NSUB * NCORE)) // B
    mesh = plsc.VectorSubcoreMesh(core_axis_name="c", subcore_axis_name="s")

    def body(aug_hbm, ad_hbm, src_hbm, dst_hbm, acc_hbm, den_hbm,
             acc_sh, den_sh, sidx, didx, rows, msg, adv, exv, dstl, sem, sem2):
        c = lax.axis_index("c")
        s = lax.axis_index("s")
        lo = c * half if split else 0
        lanes = lax.iota(jnp.int32, 16)
        col_as = jnp.full((16,), H, jnp.int32)
        zv = jnp.zeros((16,), jnp.float32)

        def z2(i, _):
            msg[i // 2, pl.ds((i % 2) * 16, 16)] = zv
            return _
        lax.fori_loop(0, B * 2, z2, None)

        def z1(i, _):
            exv[pl.ds(i * 16, 16)] = zv
            return _
        lax.fori_loop(0, B // 16, z1, None)

        for k in range(zcnt):
            r0 = pl.multiple_of(s * c16 + k * B, 8)
            pltpu.sync_copy(msg, acc_sh.at[pl.ds(r0, B)])
            pltpu.sync_copy(exv, den_sh.at[pl.ds(r0, B)])
        plsc.subcore_barrier()

        base = (s * (e_pad // NSUB)) if split else (
            (s * NCORE + c) * (e_pad // (NSUB * NCORE)))

        def blk(b, _):
            e0 = pl.multiple_of(base + b * B, B)
            i1 = pltpu.async_copy(src_hbm.at[pl.ds(e0, B)], sidx, sem)
            i2 = pltpu.async_copy(dst_hbm.at[pl.ds(e0, B)], didx, sem2)
            i1.wait()
            i2.wait()
            h1 = pltpu.async_copy(aug_hbm.at[sidx], rows, sem)
            h2 = pltpu.async_copy(ad_hbm.at[didx], adv, sem2)
            h1.wait()
            h2.wait()

            def chunk(j, _):
                d16 = didx[pl.ds(j * 16, 16)]
                gidx = e0 + j * 16 + lanes
                valid = (gidx < e_real) & (d16 >= lo) & (d16 < lo + half)
                dl = jnp.where(valid, d16 - lo, 0)
                ad16 = adv[pl.ds(j * 16, 16)]
                as16 = plsc.load_gather(rows, [j * 16 + lanes, col_as])
                z = as16 + ad16
                e = jnp.maximum(z, 0.2 * z)
                ex = jnp.where(valid, jnp.exp(e), 0.0)
                exv[pl.ds(j * 16, 16)] = ex
                dstl[pl.ds(j * 16, 16)] = dl
                return _
            lax.fori_loop(0, B // 16, chunk, None)

            def scale(i, _):
                i16 = jnp.full((16,), i, jnp.int32)
                exb = plsc.load_gather(exv, [i16])
                r0 = plsc.load_gather(rows, [i16, lanes])
                r1 = plsc.load_gather(rows, [i16, lanes + 16])
                plsc.store_scatter(msg, [i16, lanes], r0 * exb)
                plsc.store_scatter(msg, [i16, lanes + 16], r1 * exb)
                return _
            lax.fori_loop(0, B, scale, None)

            pltpu.sync_copy(msg, acc_sh.at[dstl], add=True)
            pltpu.sync_copy(exv, den_sh.at[dstl], add=True)
            return _
        lax.fori_loop(0, nblk, blk, None)

        plsc.subcore_barrier()
        for k in range(zcnt):
            r0 = pl.multiple_of(s * c16 + k * B, 8)
            pltpu.sync_copy(acc_sh.at[pl.ds(r0, B)],
                            acc_hbm.at[c, pl.ds(r0, B)])
            d0 = pl.multiple_of(c * P + s * c16 + k * B, 8)
            pltpu.sync_copy(den_sh.at[pl.ds(r0, B)], den_hbm.at[pl.ds(d0, B)])

    return pl.kernel(
        body,
        mesh=mesh,
        compiler_params=pltpu.CompilerParams(
            needs_layout_passes=False, use_tc_tiling_on_sc=False),
        out_type=(jax.ShapeDtypeStruct((NCORE, P, H), jnp.float32),
                  jax.ShapeDtypeStruct((NCORE * P,), jnp.float32)),
        scratch_types=[
            pltpu.VMEM_SHARED((P, H), jnp.float32),
            pltpu.VMEM_SHARED((P,), jnp.float32),
            pltpu.VMEM((B,), jnp.int32),
            pltpu.VMEM((B,), jnp.int32),
            pltpu.VMEM((B, AUGW), jnp.float32),
            pltpu.VMEM((B, H), jnp.float32),
            pltpu.VMEM((B,), jnp.float32),
            pltpu.VMEM((B,), jnp.float32),
            pltpu.VMEM((B,), jnp.int32),
            pltpu.SemaphoreType.DMA,
            pltpu.SemaphoreType.DMA,
        ],
    )


@functools.lru_cache(maxsize=None)
def _gather_kernel(l_pad, n_rows):
    nblk = l_pad // (NSUB * NCORE * B)
    mesh = plsc.VectorSubcoreMesh(core_axis_name="c", subcore_axis_name="s")

    def body(tab_hbm, idx_hbm, out_hbm, idxv, rowsv, sem):
        c = lax.axis_index("c")
        s = lax.axis_index("s")
        base = (s * NCORE + c) * (l_pad // (NSUB * NCORE))

        def blk(b, _):
            e0 = pl.multiple_of(base + b * B, B)
            pltpu.sync_copy(idx_hbm.at[pl.ds(e0, B)], idxv)
            pltpu.async_copy(tab_hbm.at[idxv], rowsv, sem).wait()
            pltpu.sync_copy(rowsv, out_hbm.at[pl.ds(e0, B)])
            return _
        lax.fori_loop(0, nblk, blk, None)

    return pl.kernel(
        body,
        mesh=mesh,
        compiler_params=pltpu.CompilerParams(
            needs_layout_passes=False, use_tc_tiling_on_sc=False),
        out_type=jax.ShapeDtypeStruct((l_pad, H), jnp.float32),
        scratch_types=[
            pltpu.VMEM((B,), jnp.int32),
            pltpu.VMEM((B, H), jnp.float32),
            pltpu.SemaphoreType.DMA,
        ],
    )


def _gather_rows(table, idx):
    l = idx.shape[0]
    l_pad = _cdiv(l, NSUB * NCORE * B) * (NSUB * NCORE * B)
    if l_pad != l:
        idx = jnp.concatenate([idx, jnp.zeros((l_pad - l,), idx.dtype)])
    return _gather_kernel(l_pad, table.shape[0])(table, idx)[:l]


def _rb(n):
    return 2000 if n % 2000 == 0 else 2048


def _full(shape):
    return pl.BlockSpec(shape, lambda i: tuple(0 for _ in shape))


def _enc_gru_tc(x, h0, pe, pg):
    n, br = x.shape[0], _rb(x.shape[0])

    def body(x_r, h_r, w_r, b_r, wih_r, whh_r, bih_r, bhh_r, o_r):
        xb = jnp.dot(x_r[...], w_r[...],
                     preferred_element_type=jnp.float32) + b_r[...]
        gi = jnp.dot(xb, wih_r[...],
                     preferred_element_type=jnp.float32) + bih_r[...]
        gh = jnp.dot(h_r[...], whh_r[...],
                     preferred_element_type=jnp.float32) + bhh_r[...]
        ir, iz, inn = jnp.split(gi, 3, axis=1)
        hr, hz, hn = jnp.split(gh, 3, axis=1)
        r = jax.nn.sigmoid(ir + hr)
        z = jax.nn.sigmoid(iz + hz)
        nn = jnp.tanh(inn + r * hn)
        o_r[...] = (1.0 - z) * nn + z * h_r[...]

    fi = x.shape[1]
    return pl.pallas_call(
        body,
        grid=(n // br,),
        in_specs=[pl.BlockSpec((br, fi), lambda i: (i, 0)),
                  pl.BlockSpec((br, H), lambda i: (i, 0)),
                  _full((fi, H)), _full((H,)),
                  _full((H, 3 * H)), _full((H, 3 * H)),
                  _full((3 * H,)), _full((3 * H,))],
        out_specs=pl.BlockSpec((br, H), lambda i: (i, 0)),
        out_shape=jax.ShapeDtypeStruct((n, H), jnp.float32),
    )(x, h0, pe["W"], pe["b"], pg["W_ih"], pg["W_hh"], pg["b_ih"], pg["b_hh"])


def _enc_tc(x, p):
    n, br, fi = x.shape[0], _rb(x.shape[0]), x.shape[1]

    def body(x_r, w_r, b_r, o_r):
        o_r[...] = jnp.dot(x_r[...], w_r[...],
                           preferred_element_type=jnp.float32) + b_r[...]

    return pl.pallas_call(
        body,
        grid=(n // br,),
        in_specs=[pl.BlockSpec((br, fi), lambda i: (i, 0)),
                  _full((fi, H)), _full((H,))],
        out_specs=pl.BlockSpec((br, H), lambda i: (i, 0)),
        out_shape=jax.ShapeDtypeStruct((n, H), jnp.float32),
    )(x, p["W"], p["b"])


def _aug_tc(x, w_src, a_src):
    n, br = x.shape[0], _rb(x.shape[0])

    def body(x_r, w_r, a_r, o_r):
        hs = jnp.dot(x_r[...], w_r[...], preferred_element_type=jnp.float32)
        a_s = jnp.dot(hs, a_r[...][:, None],
                      preferred_element_type=jnp.float32)
        o_r[...] = jnp.concatenate(
            [hs, a_s, jnp.zeros((br, AUGW - H - 1), jnp.float32)], axis=1)

    return pl.pallas_call(
        body,
        grid=(n // br,),
        in_specs=[pl.BlockSpec((br, H), lambda i: (i, 0)),
                  _full((H, H)), _full((H,))],
        out_specs=pl.BlockSpec((br, AUGW), lambda i: (i, 0)),
        out_shape=jax.ShapeDtypeStruct((n, AUGW), jnp.float32),
    )(x, w_src, a_src)


def _advec_tc(x, wd, av):
    n, br, r = x.shape[0], _rb(x.shape[0]), wd.shape[0]

    def body(x_r, wd_r, av_r, o_r):
        wds, avs = wd_r[...], av_r[...]
        cols = jnp.stack(
            [jnp.dot(wds[j], avs[j], preferred_element_type=jnp.float32)
             for j in range(r)], axis=1)
        o_r[...] = jnp.dot(x_r[...], cols,
                           preferred_element_type=jnp.float32)

    return pl.pallas_call(
        body,
        grid=(n // br,),
        in_specs=[pl.BlockSpec((br, H), lambda i: (i, 0)),
                  _full((r, H, H)), _full((r, H))],
        out_specs=pl.BlockSpec((br, r), lambda i: (i, 0)),
        out_shape=jax.ShapeDtypeStruct((n, r), jnp.float32),
    )(x, wd, av)


def _epilogue_tc(accs, dens, biases):
    r, n = accs[0].shape[0], accs[0].shape[1]
    br = _rb(n)
    ncopy = len(accs)
    dens = [d[..., None] for d in dens]

    def body(*refs):
        o_r = refs[-1]
        acc_v = [rf[...] for rf in refs[:ncopy]]
        den_v = [rf[...] for rf in refs[ncopy:2 * ncopy]]
        b_v = refs[2 * ncopy][...]
        out = jnp.zeros((br, H), jnp.float32)
        for j in range(r):
            a = acc_v[0][j]
            d = den_v[0][j]
            for k in range(1, ncopy):
                a = a + acc_v[k][j]
                d = d + den_v[k][j]
            out = out + a / (d + 1e-16) + b_v[j]
        o_r[...] = jax.nn.relu(out)

    return pl.pallas_call(
        body,
        grid=(n // br,),
        in_specs=([pl.BlockSpec((r, br, H), lambda i: (0, i, 0))] * ncopy
                  + [pl.BlockSpec((r, br, 1), lambda i: (0, i, 0))] * ncopy
                  + [_full((r, H))]),
        out_specs=pl.BlockSpec((br, H), lambda i: (i, 0)),
        out_shape=jax.ShapeDtypeStruct((n, H), jnp.float32),
    )(*accs, *dens, biases)


def _corr_head_tc(ag, bg, p):
    n, br = ag.shape[0], _rb(ag.shape[0])
    w1a, w1b = p["l1"]["W"][:H], p["l1"]["W"][H:]

    def body(a_r, b_r, wa_r, wb_r, b1_r, w2_r, b2_r, o_r):
        h = jax.nn.relu(
            jnp.dot(a_r[...], wa_r[...], preferred_element_type=jnp.float32)
            + jnp.dot(b_r[...], wb_r[...], preferred_element_type=jnp.float32)
            + b1_r[...])
        o_r[...] = jnp.dot(h, w2_r[...],
                           preferred_element_type=jnp.float32) + b2_r[...]

    return pl.pallas_call(
        body,
        grid=(n // br,),
        in_specs=[pl.BlockSpec((br, H), lambda i: (i, 0)),
                  pl.BlockSpec((br, H), lambda i: (i, 0)),
                  _full((H, H)), _full((H, H)), _full((H,)),
                  _full((H, 1)), _full((1,))],
        out_specs=pl.BlockSpec((br, 1), lambda i: (i, 0)),
        out_shape=jax.ShapeDtypeStruct((n, 1), jnp.float32),
    )(ag, bg, w1a, w1b, p["l1"]["b"], p["l2"]["W"], p["l2"]["b"])[:, 0]


def _mast_head_tc(xc, p):
    n, br = xc.shape[0], _rb(xc.shape[0])

    def body(x_r, w1_r, b1_r, w2_r, b2_r, o_r):
        h = jax.nn.relu(jnp.dot(x_r[...], w1_r[...],
                                preferred_element_type=jnp.float32) + b1_r[...])
        o_r[...] = jax.nn.sigmoid(
            jnp.dot(h, w2_r[...], preferred_element_type=jnp.float32)
            + b2_r[...])

    return pl.pallas_call(
        body,
        grid=(n // br,),
        in_specs=[pl.BlockSpec((br, H), lambda i: (i, 0)),
                  _full((H, H)), _full((H,)), _full((H, 1)), _full((1,))],
        out_specs=pl.BlockSpec((br, 1), lambda i: (i, 0)),
        out_shape=jax.ShapeDtypeStruct((n, 1), jnp.float32),
    )(xc, p["l1"]["W"], p["l1"]["b"], p["l2"]["W"], p["l2"]["b"])[:, 0]


def _pad_edges(src, dst):
    e = src.shape[0]
    e_pad = _cdiv(e, NSUB * NCORE * B) * (NSUB * NCORE * B)
    if e_pad != e:
        pad = e_pad - e
        src = jnp.concatenate([src, jnp.zeros((pad,), src.dtype)])
        dst = jnp.concatenate([dst, jnp.zeros((pad,), dst.dtype)])
    return src, dst, e


def _gat_sc(aug, ad, src_p, dst_p, e_real, n_dst):
    split = n_dst * H * 4 > 7 * 2**20
    half = n_dst // NCORE if split else n_dst
    P = _cdiv(half, 2048) * 2048
    k = _edge_kernel(src_p.shape[0], e_real, aug.shape[0], half, split)
    acc, den = k(aug, ad, src_p, dst_p)
    den = den.reshape(NCORE, P)
    if split:
        return ([acc[:, :half].reshape(n_dst, H)],
                [den[:, :half].reshape(n_dst)])
    return ([acc[0][:half], acc[1][:half]], [den[0][:half], den[1][:half]])


def _node_out(gats, biases):
    ncopy = len(gats[0][0])
    accs = [jnp.stack([g[0][k] for g in gats]) for k in range(ncopy)]
    dens = [jnp.stack([g[1][k] for g in gats]) for k in range(ncopy)]
    return _epilogue_tc(accs, dens, jnp.stack(biases))


def kernel(x_student, x_question, x_concept, prev_h, e_att_src, e_att_dst,
           e_tag_src, e_tag_dst, e_pre_src, e_pre_dst, e_mas_src, e_mas_dst,
           label_src, label_dst, params):
    ns, nq, nc = x_student.shape[0], x_question.shape[0], x_concept.shape[0]
    att_s, att_d, att_e = _pad_edges(e_att_src, e_att_dst)
    tag_s, tag_d, tag_e = _pad_edges(e_tag_src, e_tag_dst)
    pre_s, pre_d, pre_e = _pad_edges(e_pre_src, e_pre_dst)
    mas_s, mas_d, mas_e = _pad_edges(e_mas_src, e_mas_dst)

    xs = _enc_gru_tc(x_student, prev_h, params["enc_s"], params["gru"])
    xq = _enc_tc(x_question, params["enc_q"])
    xc = _enc_tc(x_concept, params["enc_c"])

    def hetero(conv, xs, xq, xc):
        def aug(r, x):
            return _aug_tc(x, conv[r]["W_src"], conv[r]["a_src"])

        def advec(x, rels):
            return _advec_tc(x, jnp.stack([conv[r]["W_dst"] for r in rels]),
                             jnp.stack([conv[r]["a_dst"] for r in rels]))

        ad_s = advec(xs, ["rev_att", "rev_mas"])
        ad_q = advec(xq, ["att", "rev_tag"])
        ad_c = advec(xc, ["tag", "pre", "mas"])
        s = _node_out(
            [_gat_sc(aug("rev_att", xq), ad_s[:, 0], att_d, att_s, att_e, ns),
             _gat_sc(aug("rev_mas", xc), ad_s[:, 1], mas_d, mas_s, mas_e, ns)],
            [conv["rev_att"]["bias"], conv["rev_mas"]["bias"]])
        q = _node_out(
            [_gat_sc(aug("att", xs), ad_q[:, 0], att_s, att_d, att_e, nq),
             _gat_sc(aug("rev_tag", xc), ad_q[:, 1], tag_d, tag_s, tag_e, nq)],
            [conv["att"]["bias"], conv["rev_tag"]["bias"]])
        c = _node_out(
            [_gat_sc(aug("tag", xq), ad_c[:, 0], tag_s, tag_d, tag_e, nc),
             _gat_sc(aug("pre", xc), ad_c[:, 1], pre_s, pre_d, pre_e, nc),
             _gat_sc(aug("mas", xs), ad_c[:, 2], mas_s, mas_d, mas_e, nc)],
            [conv["tag"]["bias"], conv["pre"]["bias"], conv["mas"]["bias"]])
        return s, q, c

    xs, xq, xc = hetero(params["conv1"], xs, xq, xc)
    xs, xq, xc = hetero(params["conv2"], xs, xq, xc)

    ag = _gather_rows(xs, label_src)
    bg = _gather_rows(xq, label_dst)
    logits = _corr_head_tc(ag, bg, params["head_corr"])
    mastery = _mast_head_tc(xc, params["head_mast"])
    return ({"student": xs, "question": xq, "concept": xc}, logits, mastery)
